# baseline (speedup 1.0000x reference)
import jax
import jax.numpy as jnp
from jax import lax
from jax.experimental import pallas as pl
from jax.experimental.pallas import tpu as pltpu
from jax.experimental.pallas import tpu_sc as plsc

_K = 10
_N = 8192
_E = 50000


_R = 128


def _oem_sort_network(n):
    comps = []

    def oddeven_merge(lo, n2, r):
        step = r * 2
        if step < n2:
            oddeven_merge(lo, n2, step)
            oddeven_merge(lo + r, n2, step)
            for i in range(lo + r, lo + n2 - r, step):
                comps.append((i, i + r))
        else:
            comps.append((lo, lo + r))

    def sort_range(lo, hi):
        if hi - lo >= 1:
            mid = lo + (hi - lo) // 2
            sort_range(lo, mid)
            sort_range(mid + 1, hi)
            oddeven_merge(lo, hi - lo + 1, 1)

    sort_range(0, n - 1)
    return comps


def _pruned_topk_network(n, k):
    needed = set(range(k))
    ops = []
    for (i, j) in reversed(_oem_sort_network(n)):
        ni, nj = i in needed, j in needed
        if not (ni or nj):
            continue
        ops.append((i, j, ni, nj))
        needed.add(i)
        needed.add(j)
    return ops[::-1]


_NET = _pruned_topk_network(_N // 128, _K + 1)


def _dtm_body(pts_blk_ref, ptsT_ref, out_ref, g_ref):
    bt = ptsT_ref[...]
    a = pts_blk_ref[...]
    sqc = jnp.sum(bt * bt, axis=0, keepdims=True)
    G = jnp.dot(
        a.astype(jnp.bfloat16),
        bt.astype(jnp.bfloat16),
        preferred_element_type=jnp.float32,
    )
    g_ref[...] = sqc - 2.0 * G
    lane = lax.broadcasted_iota(jnp.int32, (8, 128), 1)
    INF = jnp.float32(jnp.inf)

    def sub_body(sub, carry):
        arow = pts_blk_ref[pl.ds(sub * 8, 8), :]
        sq_s = jnp.sum(arow * arow, axis=1, keepdims=True)
        vals = [
            sq_s + g_ref[pl.ds(sub * 8, 8), pl.ds(v * 128, 128)]
            for v in range(_N // 128)
        ]
        for (i, j, ni, nj) in _NET:
            x, y = vals[i], vals[j]
            if ni:
                vals[i] = jnp.minimum(x, y)
            if nj:
                vals[j] = jnp.maximum(x, y)
        s = vals[: _K + 1]
        acc = jnp.zeros((8, 1), jnp.float32)
        for t in range(_K + 1):
            m = jnp.min(s[0], axis=1, keepdims=True)
            if t > 0:
                acc = acc + jnp.maximum(m, 1e-12)
            if t < _K:
                hit = s[0] == m
                li = jnp.min(
                    jnp.where(hit, lane, jnp.int32(999)), axis=1, keepdims=True
                )
                hf = lane == li
                for i2 in range(_K):
                    s[i2] = jnp.where(hf, s[i2 + 1], s[i2])
                s[_K] = jnp.where(hf, INF, s[_K])
        out_ref[pl.ds(sub * 8, 8), :] = jnp.sqrt(acc * (1.0 / _K))
        return carry

    lax.fori_loop(0, _R // 8, sub_body, 0)


def _dtm(pts):
    out = pl.pallas_call(
        _dtm_body,
        grid=(_N // _R,),
        in_specs=[
            pl.BlockSpec((_R, 3), lambda i: (i, 0)),
            pl.BlockSpec((3, _N), lambda i: (0, 0)),
        ],
        out_specs=pl.BlockSpec((_R, 1), lambda i: (i, 0)),
        out_shape=jax.ShapeDtypeStruct((_N, 1), jnp.float32),
        scratch_shapes=[pltpu.VMEM((_R, _N), jnp.float32)],
    )(pts, pts.T)
    return out[:, 0]


_NW = 32
_EPW = 1568
_EPAD = _NW * _EPW


def _edge_body(tbl_hbm, eu_hbm, ev_hbm, out_hbm, tbl_v, iu_v, iv_v, res_v):
    c = lax.axis_index("c")
    s = lax.axis_index("s")
    wid = s * 2 + c
    base = wid * _EPW
    pltpu.sync_copy(tbl_hbm, tbl_v)
    pltpu.sync_copy(eu_hbm.at[pl.ds(base, _EPW)], iu_v)
    pltpu.sync_copy(ev_hbm.at[pl.ds(base, _EPW)], iv_v)
    for i in range(_EPW // 16):
        u = iu_v[pl.ds(i * 16, 16)]
        v = iv_v[pl.ds(i * 16, 16)]
        xu = plsc.load_gather(tbl_v, [u])
        xv = plsc.load_gather(tbl_v, [v])
        yu = plsc.load_gather(tbl_v, [u + _N])
        yv = plsc.load_gather(tbl_v, [v + _N])
        zu = plsc.load_gather(tbl_v, [u + 2 * _N])
        zv = plsc.load_gather(tbl_v, [v + 2 * _N])
        fu = plsc.load_gather(tbl_v, [u + 3 * _N])
        fv = plsc.load_gather(tbl_v, [v + 3 * _N])
        dx = xu - xv
        dy = yu - yv
        dz = zu - zv
        s2 = dx * dx + dy * dy + dz * dz + 1e-12
        ib = plsc.bitcast(s2, jnp.int32)
        yb = lax.shift_right_logical(ib, jnp.int32(1)) + jnp.int32(0x1FBD1DF5)
        y = plsc.bitcast(yb, jnp.float32)
        y = 0.5 * (y + s2 / y)
        y = 0.5 * (y + s2 / y)
        y = 0.5 * (y + s2 / y)
        res_v[pl.ds(i * 16, 16)] = y + jnp.maximum(fu, fv)
    pltpu.sync_copy(res_v, out_hbm.at[pl.ds(base, _EPW)])


def _edge_call(tbl, eu, ev):
    mesh = plsc.VectorSubcoreMesh(core_axis_name="c", subcore_axis_name="s")
    run = pl.kernel(
        _edge_body,
        out_type=jax.ShapeDtypeStruct((_EPAD,), jnp.float32),
        mesh=mesh,
        compiler_params=pltpu.CompilerParams(needs_layout_passes=False),
        scratch_types=[
            pltpu.VMEM((4 * _N,), jnp.float32),
            pltpu.VMEM((_EPW,), jnp.int32),
            pltpu.VMEM((_EPW,), jnp.int32),
            pltpu.VMEM((_EPW,), jnp.float32),
        ],
    )
    return run(tbl, eu, ev)


def kernel(pts, edges):
    dtm = _dtm(pts)
    tbl = jnp.concatenate([pts, dtm[:, None]], axis=1).T.reshape(-1)
    eu = jnp.zeros((_EPAD,), jnp.int32).at[:_E].set(edges[:, 0])
    ev = jnp.zeros((_EPAD,), jnp.int32).at[:_E].set(edges[:, 1])
    out = _edge_call(tbl, eu, ev)
    return out[:_E]


# speedup vs baseline: 7.6399x; 7.6399x over previous
import jax
import jax.numpy as jnp
from jax import lax
from jax.experimental import pallas as pl
from jax.experimental.pallas import tpu as pltpu
from jax.experimental.pallas import tpu_sc as plsc

_K = 10
_N = 8192
_E = 50000


_R = 256


def _oem_sort_network(n):
    comps = []

    def oddeven_merge(lo, n2, r):
        step = r * 2
        if step < n2:
            oddeven_merge(lo, n2, step)
            oddeven_merge(lo + r, n2, step)
            for i in range(lo + r, lo + n2 - r, step):
                comps.append((i, i + r))
        else:
            comps.append((lo, lo + r))

    def sort_range(lo, hi):
        if hi - lo >= 1:
            mid = lo + (hi - lo) // 2
            sort_range(lo, mid)
            sort_range(mid + 1, hi)
            oddeven_merge(lo, hi - lo + 1, 1)

    sort_range(0, n - 1)
    return comps


def _pruned_topk_network(n, k):
    needed = set(range(k))
    ops = []
    for (i, j) in reversed(_oem_sort_network(n)):
        ni, nj = i in needed, j in needed
        if not (ni or nj):
            continue
        ops.append((i, j, ni, nj))
        needed.add(i)
        needed.add(j)
    return ops[::-1]


_NET = _pruned_topk_network(_N // 128, _K + 1)


def _dtm_body(pts_blk_ref, ptsT_ref, out_ref):
    bt = ptsT_ref[...]
    a = pts_blk_ref[...]
    sqr = jnp.sum(a * a, axis=1, keepdims=True)
    sqc = jnp.sum(bt * bt, axis=0, keepdims=True)
    G = jnp.dot(
        a.astype(jnp.bfloat16),
        bt.astype(jnp.bfloat16),
        preferred_element_type=jnp.float32,
    )
    R = a.shape[0]
    INF = jnp.float32(jnp.inf)
    s = [jnp.full((R, 128), INF, jnp.float32) for _ in range(_K + 1)]
    for v in range(_N // 128):
        sl = slice(v * 128, (v + 1) * 128)
        Dv = (sqr + sqc[:, sl]) - 2.0 * G[:, sl]
        mx = [jnp.maximum(s[i], Dv) for i in range(_K)]
        s[0] = jnp.minimum(s[0], Dv)
        for i in range(1, _K + 1):
            s[i] = jnp.minimum(s[i], mx[i - 1])
    lane = lax.broadcasted_iota(jnp.int32, (R, 128), 1)
    acc = jnp.zeros((R, 1), jnp.float32)
    for t in range(_K + 1):
        m = jnp.min(s[0], axis=1, keepdims=True)
        if t > 0:
            acc = acc + jnp.maximum(m, 1e-12)
        if t < _K:
            hit = s[0] == m
            li = jnp.min(
                jnp.where(hit, lane, jnp.int32(999)), axis=1, keepdims=True
            )
            hf = lane == li
            for i in range(_K):
                s[i] = jnp.where(hf, s[i + 1], s[i])
            s[_K] = jnp.where(hf, INF, s[_K])
    out_ref[...] = jnp.sqrt(acc * (1.0 / _K))


def _dtm(pts):
    out = pl.pallas_call(
        _dtm_body,
        grid=(_N // _R,),
        in_specs=[
            pl.BlockSpec((_R, 3), lambda i: (i, 0)),
            pl.BlockSpec((3, _N), lambda i: (0, 0)),
        ],
        out_specs=pl.BlockSpec((_R, 1), lambda i: (i, 0)),
        out_shape=jax.ShapeDtypeStruct((_N, 1), jnp.float32),
    )(pts, pts.T)
    return out[:, 0]


_NW = 32
_EPW = 1568
_EPAD = _NW * _EPW


def _edge_body(tbl_hbm, eu_hbm, ev_hbm, out_hbm, tbl_v, iu_v, iv_v, res_v):
    c = lax.axis_index("c")
    s = lax.axis_index("s")
    wid = s * 2 + c
    base = wid * _EPW
    pltpu.sync_copy(tbl_hbm, tbl_v)
    pltpu.sync_copy(eu_hbm.at[pl.ds(base, _EPW)], iu_v)
    pltpu.sync_copy(ev_hbm.at[pl.ds(base, _EPW)], iv_v)
    for i in range(_EPW // 16):
        u = iu_v[pl.ds(i * 16, 16)]
        v = iv_v[pl.ds(i * 16, 16)]
        xu = plsc.load_gather(tbl_v, [u])
        xv = plsc.load_gather(tbl_v, [v])
        yu = plsc.load_gather(tbl_v, [u + _N])
        yv = plsc.load_gather(tbl_v, [v + _N])
        zu = plsc.load_gather(tbl_v, [u + 2 * _N])
        zv = plsc.load_gather(tbl_v, [v + 2 * _N])
        fu = plsc.load_gather(tbl_v, [u + 3 * _N])
        fv = plsc.load_gather(tbl_v, [v + 3 * _N])
        dx = xu - xv
        dy = yu - yv
        dz = zu - zv
        s2 = dx * dx + dy * dy + dz * dz + 1e-12
        ib = plsc.bitcast(s2, jnp.int32)
        yb = lax.shift_right_logical(ib, jnp.int32(1)) + jnp.int32(0x1FBD1DF5)
        y = plsc.bitcast(yb, jnp.float32)
        y = 0.5 * (y + s2 / y)
        y = 0.5 * (y + s2 / y)
        y = 0.5 * (y + s2 / y)
        res_v[pl.ds(i * 16, 16)] = y + jnp.maximum(fu, fv)
    pltpu.sync_copy(res_v, out_hbm.at[pl.ds(base, _EPW)])


def _edge_call(tbl, eu, ev):
    mesh = plsc.VectorSubcoreMesh(core_axis_name="c", subcore_axis_name="s")
    run = pl.kernel(
        _edge_body,
        out_type=jax.ShapeDtypeStruct((_EPAD,), jnp.float32),
        mesh=mesh,
        compiler_params=pltpu.CompilerParams(needs_layout_passes=False),
        scratch_types=[
            pltpu.VMEM((4 * _N,), jnp.float32),
            pltpu.VMEM((_EPW,), jnp.int32),
            pltpu.VMEM((_EPW,), jnp.int32),
            pltpu.VMEM((_EPW,), jnp.float32),
        ],
    )
    return run(tbl, eu, ev)


def kernel(pts, edges):
    dtm = _dtm(pts)
    tbl = jnp.concatenate([pts, dtm[:, None]], axis=1).T.reshape(-1)
    eu = jnp.zeros((_EPAD,), jnp.int32).at[:_E].set(edges[:, 0])
    ev = jnp.zeros((_EPAD,), jnp.int32).at[:_E].set(edges[:, 1])
    out = _edge_call(tbl, eu, ev)
    return out[:_E]


# speedup vs baseline: 7.9804x; 1.0446x over previous
import jax
import jax.numpy as jnp
from jax import lax
from jax.experimental import pallas as pl
from jax.experimental.pallas import tpu as pltpu
from jax.experimental.pallas import tpu_sc as plsc

_K = 10
_N = 8192
_E = 50000


_R = 512


def _oem_sort_network(n):
    comps = []

    def oddeven_merge(lo, n2, r):
        step = r * 2
        if step < n2:
            oddeven_merge(lo, n2, step)
            oddeven_merge(lo + r, n2, step)
            for i in range(lo + r, lo + n2 - r, step):
                comps.append((i, i + r))
        else:
            comps.append((lo, lo + r))

    def sort_range(lo, hi):
        if hi - lo >= 1:
            mid = lo + (hi - lo) // 2
            sort_range(lo, mid)
            sort_range(mid + 1, hi)
            oddeven_merge(lo, hi - lo + 1, 1)

    sort_range(0, n - 1)
    return comps


def _pruned_topk_network(n, k):
    needed = set(range(k))
    ops = []
    for (i, j) in reversed(_oem_sort_network(n)):
        ni, nj = i in needed, j in needed
        if not (ni or nj):
            continue
        ops.append((i, j, ni, nj))
        needed.add(i)
        needed.add(j)
    return ops[::-1]


_NET = _pruned_topk_network(_N // 128, _K + 1)


def _dtm_body(pts_blk_ref, ptsT_ref, out_ref):
    bt = ptsT_ref[...]
    a = pts_blk_ref[...]
    sqr = jnp.sum(a * a, axis=1, keepdims=True)
    sqc = jnp.sum(bt * bt, axis=0, keepdims=True)
    G = jnp.dot(
        a.astype(jnp.bfloat16),
        bt.astype(jnp.bfloat16),
        preferred_element_type=jnp.float32,
    )
    R = a.shape[0]
    INF = jnp.float32(jnp.inf)
    s = [jnp.full((R, 128), INF, jnp.float32) for _ in range(_K + 1)]
    for v in range(_N // 128):
        sl = slice(v * 128, (v + 1) * 128)
        Dv = (sqr + sqc[:, sl]) - 2.0 * G[:, sl]
        mx = [jnp.maximum(s[i], Dv) for i in range(_K)]
        s[0] = jnp.minimum(s[0], Dv)
        for i in range(1, _K + 1):
            s[i] = jnp.minimum(s[i], mx[i - 1])
    lane = lax.broadcasted_iota(jnp.int32, (R, 128), 1)
    acc = jnp.zeros((R, 1), jnp.float32)
    for t in range(_K + 1):
        m = jnp.min(s[0], axis=1, keepdims=True)
        if t > 0:
            acc = acc + jnp.maximum(m, 1e-12)
        if t < _K:
            hit = s[0] == m
            li = jnp.min(
                jnp.where(hit, lane, jnp.int32(999)), axis=1, keepdims=True
            )
            hf = lane == li
            for i in range(_K):
                s[i] = jnp.where(hf, s[i + 1], s[i])
            s[_K] = jnp.where(hf, INF, s[_K])
    out_ref[...] = jnp.sqrt(acc * (1.0 / _K))


def _dtm(pts):
    out = pl.pallas_call(
        _dtm_body,
        grid=(_N // _R,),
        in_specs=[
            pl.BlockSpec((_R, 3), lambda i: (i, 0)),
            pl.BlockSpec((3, _N), lambda i: (0, 0)),
        ],
        out_specs=pl.BlockSpec((_R, 1), lambda i: (i, 0)),
        out_shape=jax.ShapeDtypeStruct((_N, 1), jnp.float32),
    )(pts, pts.T)
    return out[:, 0]


_NW = 32
_EPW = 1568
_EPAD = _NW * _EPW


def _edge_body(tbl_hbm, eu_hbm, ev_hbm, out_hbm, tbl_v, iu_v, iv_v, res_v):
    c = lax.axis_index("c")
    s = lax.axis_index("s")
    wid = s * 2 + c
    base = wid * _EPW
    pltpu.sync_copy(tbl_hbm, tbl_v)
    pltpu.sync_copy(eu_hbm.at[pl.ds(base, _EPW)], iu_v)
    pltpu.sync_copy(ev_hbm.at[pl.ds(base, _EPW)], iv_v)
    for i in range(_EPW // 16):
        u = iu_v[pl.ds(i * 16, 16)]
        v = iv_v[pl.ds(i * 16, 16)]
        xu = plsc.load_gather(tbl_v, [u])
        xv = plsc.load_gather(tbl_v, [v])
        yu = plsc.load_gather(tbl_v, [u + _N])
        yv = plsc.load_gather(tbl_v, [v + _N])
        zu = plsc.load_gather(tbl_v, [u + 2 * _N])
        zv = plsc.load_gather(tbl_v, [v + 2 * _N])
        fu = plsc.load_gather(tbl_v, [u + 3 * _N])
        fv = plsc.load_gather(tbl_v, [v + 3 * _N])
        dx = xu - xv
        dy = yu - yv
        dz = zu - zv
        s2 = dx * dx + dy * dy + dz * dz + 1e-12
        ib = plsc.bitcast(s2, jnp.int32)
        yb = lax.shift_right_logical(ib, jnp.int32(1)) + jnp.int32(0x1FBD1DF5)
        y = plsc.bitcast(yb, jnp.float32)
        y = 0.5 * (y + s2 / y)
        y = 0.5 * (y + s2 / y)
        y = 0.5 * (y + s2 / y)
        res_v[pl.ds(i * 16, 16)] = y + jnp.maximum(fu, fv)
    pltpu.sync_copy(res_v, out_hbm.at[pl.ds(base, _EPW)])


def _edge_call(tbl, eu, ev):
    mesh = plsc.VectorSubcoreMesh(core_axis_name="c", subcore_axis_name="s")
    run = pl.kernel(
        _edge_body,
        out_type=jax.ShapeDtypeStruct((_EPAD,), jnp.float32),
        mesh=mesh,
        compiler_params=pltpu.CompilerParams(needs_layout_passes=False),
        scratch_types=[
            pltpu.VMEM((4 * _N,), jnp.float32),
            pltpu.VMEM((_EPW,), jnp.int32),
            pltpu.VMEM((_EPW,), jnp.int32),
            pltpu.VMEM((_EPW,), jnp.float32),
        ],
    )
    return run(tbl, eu, ev)


def kernel(pts, edges):
    dtm = _dtm(pts)
    tbl = jnp.concatenate([pts, dtm[:, None]], axis=1).T.reshape(-1)
    eu = jnp.zeros((_EPAD,), jnp.int32).at[:_E].set(edges[:, 0])
    ev = jnp.zeros((_EPAD,), jnp.int32).at[:_E].set(edges[:, 1])
    out = _edge_call(tbl, eu, ev)
    return out[:_E]


# speedup vs baseline: 9.8127x; 1.2296x over previous
import jax
import jax.numpy as jnp
from jax import lax
from jax.experimental import pallas as pl
from jax.experimental.pallas import tpu as pltpu
from jax.experimental.pallas import tpu_sc as plsc

_K = 10
_N = 8192
_E = 50000


_R = 512


def _oem_sort_network(n):
    comps = []

    def oddeven_merge(lo, n2, r):
        step = r * 2
        if step < n2:
            oddeven_merge(lo, n2, step)
            oddeven_merge(lo + r, n2, step)
            for i in range(lo + r, lo + n2 - r, step):
                comps.append((i, i + r))
        else:
            comps.append((lo, lo + r))

    def sort_range(lo, hi):
        if hi - lo >= 1:
            mid = lo + (hi - lo) // 2
            sort_range(lo, mid)
            sort_range(mid + 1, hi)
            oddeven_merge(lo, hi - lo + 1, 1)

    sort_range(0, n - 1)
    return comps


def _pruned_topk_network(n, k):
    needed = set(range(k))
    ops = []
    for (i, j) in reversed(_oem_sort_network(n)):
        ni, nj = i in needed, j in needed
        if not (ni or nj):
            continue
        ops.append((i, j, ni, nj))
        needed.add(i)
        needed.add(j)
    return ops[::-1]


_NET = _pruned_topk_network(_N // 128, _K + 1)


def _dtm_body(pts_blk_ref, ptsT_ref, out_ref):
    bt = ptsT_ref[...]
    a = pts_blk_ref[...]
    sqr = jnp.sum(a * a, axis=1, keepdims=True)
    sqc = jnp.sum(bt * bt, axis=0, keepdims=True)
    G = jnp.dot(
        a.astype(jnp.bfloat16),
        bt.astype(jnp.bfloat16),
        preferred_element_type=jnp.float32,
    )
    Gp = sqc - 2.0 * G
    R = a.shape[0]
    INF = jnp.float32(jnp.inf)

    def insert(lst, x):
        mx = [jnp.maximum(lst[i], x) for i in range(len(lst) - 1)]
        lst[0] = jnp.minimum(lst[0], x)
        for i in range(1, len(lst)):
            lst[i] = jnp.minimum(lst[i], mx[i - 1])

    s = [jnp.full((R, 128), INF, jnp.float32) for _ in range(_K + 1)]
    h = [jnp.full((R, 128), INF, jnp.float32) for _ in range(5)]
    q = [jnp.full((R, 128), INF, jnp.float32) for _ in range(3)]
    r = [jnp.full((R, 128), INF, jnp.float32) for _ in range(2)]
    for v in range(0, _N // 128, 4):
        d = [sqr + Gp[:, (v + u) * 128:(v + u + 1) * 128] for u in range(4)]
        for (i, j) in ((0, 1), (2, 3), (0, 2), (1, 3), (1, 2)):
            lo, hi = jnp.minimum(d[i], d[j]), jnp.maximum(d[i], d[j])
            d[i], d[j] = lo, hi
        insert(s, d[0])
        insert(h, d[1])
        insert(q, d[2])
        insert(r, d[3])
    for x in h + q + r:
        insert(s, x)
    lane = lax.broadcasted_iota(jnp.int32, (R, 128), 1)
    acc = jnp.zeros((R, 1), jnp.float32)
    for t in range(_K + 1):
        m = jnp.min(s[0], axis=1, keepdims=True)
        if t > 0:
            acc = acc + jnp.maximum(m, 1e-12)
        if t < _K:
            hit = s[0] == m
            li = jnp.min(
                jnp.where(hit, lane, jnp.int32(999)), axis=1, keepdims=True
            )
            hf = lane == li
            for i in range(_K):
                s[i] = jnp.where(hf, s[i + 1], s[i])
            s[_K] = jnp.where(hf, INF, s[_K])
    out_ref[...] = jnp.sqrt(acc * (1.0 / _K))


def _dtm(pts):
    out = pl.pallas_call(
        _dtm_body,
        grid=(_N // _R,),
        in_specs=[
            pl.BlockSpec((_R, 3), lambda i: (i, 0)),
            pl.BlockSpec((3, _N), lambda i: (0, 0)),
        ],
        out_specs=pl.BlockSpec((_R, 1), lambda i: (i, 0)),
        out_shape=jax.ShapeDtypeStruct((_N, 1), jnp.float32),
    )(pts, pts.T)
    return out[:, 0]


_NW = 32
_EPW = 1568
_EPAD = _NW * _EPW


def _edge_body(tbl_hbm, eu_hbm, ev_hbm, out_hbm, tbl_v, iu_v, iv_v, res_v):
    c = lax.axis_index("c")
    s = lax.axis_index("s")
    wid = s * 2 + c
    base = wid * _EPW
    pltpu.sync_copy(tbl_hbm, tbl_v)
    pltpu.sync_copy(eu_hbm.at[pl.ds(base, _EPW)], iu_v)
    pltpu.sync_copy(ev_hbm.at[pl.ds(base, _EPW)], iv_v)
    for i in range(_EPW // 16):
        u = iu_v[pl.ds(i * 16, 16)]
        v = iv_v[pl.ds(i * 16, 16)]
        xu = plsc.load_gather(tbl_v, [u])
        xv = plsc.load_gather(tbl_v, [v])
        yu = plsc.load_gather(tbl_v, [u + _N])
        yv = plsc.load_gather(tbl_v, [v + _N])
        zu = plsc.load_gather(tbl_v, [u + 2 * _N])
        zv = plsc.load_gather(tbl_v, [v + 2 * _N])
        fu = plsc.load_gather(tbl_v, [u + 3 * _N])
        fv = plsc.load_gather(tbl_v, [v + 3 * _N])
        dx = xu - xv
        dy = yu - yv
        dz = zu - zv
        s2 = dx * dx + dy * dy + dz * dz + 1e-12
        ib = plsc.bitcast(s2, jnp.int32)
        yb = lax.shift_right_logical(ib, jnp.int32(1)) + jnp.int32(0x1FBD1DF5)
        y = plsc.bitcast(yb, jnp.float32)
        y = 0.5 * (y + s2 / y)
        y = 0.5 * (y + s2 / y)
        y = 0.5 * (y + s2 / y)
        res_v[pl.ds(i * 16, 16)] = y + jnp.maximum(fu, fv)
    pltpu.sync_copy(res_v, out_hbm.at[pl.ds(base, _EPW)])


def _edge_call(tbl, eu, ev):
    mesh = plsc.VectorSubcoreMesh(core_axis_name="c", subcore_axis_name="s")
    run = pl.kernel(
        _edge_body,
        out_type=jax.ShapeDtypeStruct((_EPAD,), jnp.float32),
        mesh=mesh,
        compiler_params=pltpu.CompilerParams(needs_layout_passes=False),
        scratch_types=[
            pltpu.VMEM((4 * _N,), jnp.float32),
            pltpu.VMEM((_EPW,), jnp.int32),
            pltpu.VMEM((_EPW,), jnp.int32),
            pltpu.VMEM((_EPW,), jnp.float32),
        ],
    )
    return run(tbl, eu, ev)


def kernel(pts, edges):
    dtm = _dtm(pts)
    tbl = jnp.concatenate([pts, dtm[:, None]], axis=1).T.reshape(-1)
    eu = jnp.zeros((_EPAD,), jnp.int32).at[:_E].set(edges[:, 0])
    ev = jnp.zeros((_EPAD,), jnp.int32).at[:_E].set(edges[:, 1])
    out = _edge_call(tbl, eu, ev)
    return out[:_E]
